# baseline (speedup 1.0000x reference)
import functools

import jax
import jax.numpy as jnp
import numpy as np
from jax.experimental import pallas as pl
from jax.experimental.pallas import tpu as pltpu

B, S, H, L, NH, DH = 1, 2048, 768, 2, 12, 64
V, TV, FF, BS, NR = 30522, 2, 3072, 64, 3
NB = S // BS
NK = 5 + NR


def _block_layout(nb, num_rand, seed):
    rng = np.random.RandomState(seed)
    idx = np.zeros((nb, 5 + num_rand), np.int32)
    valid = np.ones((nb, 5 + num_rand), np.float32)
    for i in range(nb):
        fixed = [0, nb - 1, (i - 1) % nb, i, (i + 1) % nb]
        rem = sorted(set(range(nb)) - set(fixed))
        r = rng.choice(rem, num_rand, replace=False)
        row = fixed + list(r)
        seen = set()
        for j, c in enumerate(row):
            idx[i, j] = c
            if c in seen:
                valid[i, j] = 0.0
            seen.add(c)
    return idx, valid


_LAYOUTS = [_block_layout(NB, NR, i) for i in range(L)]

def _ln(x, g, b):
    m = jnp.mean(x, -1, keepdims=True)
    v = jnp.mean((x - m) * (x - m), -1, keepdims=True)
    return (x - m) / jnp.sqrt(v + 1e-12) * g + b


_QB = 512
_NQ = S // _QB
_AB = 2
_NA = NB // _AB
_STEPS = _NQ + _NA + _NQ


def _layer_body(embed, *refs):
    if embed:
        (idx_ref, val_ref, ids_ref, tab_ref, pos_ref, tid_ref, te_ref,
         ge_ref, be_ref, wqkv_ref, bqkv_ref, mask_ref, wo_ref, bo_ref,
         g1_ref, b1_ref, w1_ref, bb1_ref, w2_ref, bb2_ref, g2_ref, b2_ref,
         out_ref, q_scr, kt_scr, v_scr, o_scr, kgt_scr, vg_scr,
         x_scr, sem) = refs
    else:
        (idx_ref, val_ref, x_in_ref, wqkv_ref, bqkv_ref, mask_ref,
         wo_ref, bo_ref, g1_ref, b1_ref, w1_ref, bb1_ref, w2_ref, bb2_ref,
         g2_ref, b2_ref, out_ref, q_scr, kt_scr, v_scr, o_scr, kgt_scr,
         vg_scr) = refs
    step = pl.program_id(0)

    if embed:
        @pl.when(step == 0)
        def _():
            def issue(t, _):
                for u in range(8):
                    pltpu.make_async_copy(
                        tab_ref.at[pl.ds(ids_ref[t * 8 + u], 1), :],
                        x_scr.at[pl.ds(t * 8 + u, 1), :],
                        sem,
                    ).start()
                return 0

            jax.lax.fori_loop(0, S // 8, issue, 0)

    @pl.when(step < _NQ)
    def _():
        i = step
        if embed:
            def wait16(t, _):
                for _u in range(16):
                    pltpu.make_async_copy(
                        tab_ref.at[pl.ds(0, 1), :],
                        x_scr.at[pl.ds(0, 1), :], sem,
                    ).wait()
                return 0

            jax.lax.fori_loop(0, _QB // 16, wait16, 0)
            raw = x_scr[pl.ds(i * _QB, _QB), :] + pos_ref[...]
            cond = tid_ref[...] == 0
            raw = raw + jnp.where(cond, te_ref[0:1, :], te_ref[1:2, :])
            xb_f32 = _ln(raw, ge_ref[...], be_ref[...])
            x_scr[pl.ds(i * _QB, _QB), :] = xb_f32
        else:
            xb_f32 = x_in_ref[...]
        xb = xb_f32.astype(jnp.bfloat16)
        r = jax.lax.dot_general(
            xb, wqkv_ref[...], (((1,), (0,)), ((), ())),
            preferred_element_type=jnp.float32)
        r = (r + bqkv_ref[...]).astype(jnp.bfloat16)
        for h in range(NH):
            q_scr[h, pl.ds(i * _QB, _QB), :] = r[:, h * DH:(h + 1) * DH]
            v_scr[h, pl.ds(i * _QB, _QB), :] = (
                r[:, 2 * H + h * DH:2 * H + (h + 1) * DH])
            for sb in range(_QB // BS):
                kt_scr[pl.ds(i * (_QB // BS) + sb, 1), h] = jnp.transpose(
                    r[sb * BS:(sb + 1) * BS,
                      H + h * DH:H + (h + 1) * DH])[None]

    @pl.when(jnp.logical_and(step >= _NQ, step < _NQ + _NA))
    def _():
        for local in range(_AB):
            n = (step - _NQ) * _AB + local
            bias_parts = []
            for j in range(NK):
                bi = idx_ref[n * NK + j]
                kgt_scr[local, :, :, j * BS:(j + 1) * BS] = kt_scr[bi]
                vg_scr[local, :, j * BS:(j + 1) * BS, :] = (
                    v_scr[:, pl.ds(bi * BS, BS), :])
                mv = mask_ref[bi]
                vj = val_ref[n * NK + j].astype(jnp.float32)
                bias_parts.append((1.0 - mv * vj) * (-1e9))
            bias = jnp.concatenate(bias_parts, axis=-1)[None]

            qb = q_scr[:, pl.ds(n * BS, BS), :]
            s = jax.lax.dot_general(
                qb, kgt_scr[local], (((2,), (1,)), ((0,), (0,))),
                preferred_element_type=jnp.float32)
            e = jnp.exp(s * 0.125 + bias)
            denom = jnp.sum(e, -1, keepdims=True)
            o3 = jax.lax.dot_general(
                e.astype(jnp.bfloat16), vg_scr[local],
                (((2,), (1,)), ((0,), (0,))),
                preferred_element_type=jnp.float32)
            o3 = o3 * (1.0 / denom)
            for h in range(NH):
                o_scr[pl.ds(n * BS, BS), h * DH:(h + 1) * DH] = (
                    o3[h].astype(jnp.bfloat16))

    @pl.when(step >= _NQ + _NA)
    def _():
        i = step - (_NQ + _NA)
        ob = o_scr[pl.ds(i * _QB, _QB), :]
        a = jax.lax.dot_general(
            ob, wo_ref[...], (((1,), (0,)), ((), ())),
            preferred_element_type=jnp.float32)
        if embed:
            xres = x_scr[pl.ds(i * _QB, _QB), :]
        else:
            xres = x_in_ref[...]
        a = a + bo_ref[...] + xres
        x1 = _ln(a, g1_ref[...], b1_ref[...])
        h1 = jax.lax.dot_general(
            x1.astype(jnp.bfloat16), w1_ref[...], (((1,), (0,)), ((), ())),
            preferred_element_type=jnp.float32)
        h1 = jax.nn.gelu(h1 + bb1_ref[...])
        f = jax.lax.dot_general(
            h1.astype(jnp.bfloat16), w2_ref[...], (((1,), (0,)), ((), ())),
            preferred_element_type=jnp.float32)
        f = f + bb2_ref[...] + x1
        out_ref[...] = _ln(f, g2_ref[...], b2_ref[...])


def _xmap(s, *_):
    return (jnp.where(s < _NQ, s,
                      jnp.where(s >= _NQ + _NA, s - (_NQ + _NA), 0)), 0)


_WSPECS = [
    pl.BlockSpec((H, 3 * H), lambda s, *_: (0, 0)),
    pl.BlockSpec((1, 3 * H), lambda s, *_: (0, 0)),
    pl.BlockSpec((NB, 1, BS), lambda s, *_: (0, 0, 0)),
    pl.BlockSpec((H, H), lambda s, *_: (0, 0)),
    pl.BlockSpec((1, H), lambda s, *_: (0, 0)),
    pl.BlockSpec((1, H), lambda s, *_: (0, 0)),
    pl.BlockSpec((1, H), lambda s, *_: (0, 0)),
    pl.BlockSpec((H, FF), lambda s, *_: (0, 0)),
    pl.BlockSpec((1, FF), lambda s, *_: (0, 0)),
    pl.BlockSpec((FF, H), lambda s, *_: (0, 0)),
    pl.BlockSpec((1, H), lambda s, *_: (0, 0)),
    pl.BlockSpec((1, H), lambda s, *_: (0, 0)),
    pl.BlockSpec((1, H), lambda s, *_: (0, 0)),
]

_OUT_SPEC_F = lambda s, *_: (jnp.where(s >= _NQ + _NA, s - (_NQ + _NA), 0), 0)

_SCRATCH = [
    pltpu.VMEM((NH, S, DH), jnp.bfloat16),
    pltpu.VMEM((NB, NH, DH, BS), jnp.bfloat16),
    pltpu.VMEM((NH, S, DH), jnp.bfloat16),
    pltpu.VMEM((S, H), jnp.bfloat16),
    pltpu.VMEM((_AB, NH, DH, NK * BS), jnp.bfloat16),
    pltpu.VMEM((_AB, NH, NK * BS, DH), jnp.bfloat16),
]


def _layer(x, wqkv, bqkv, mask_f, idx_flat, val_flat, *weights):
    grid_spec = pltpu.PrefetchScalarGridSpec(
        num_scalar_prefetch=2,
        grid=(_STEPS,),
        in_specs=[pl.BlockSpec((_QB, H), _xmap)] + _WSPECS,
        out_specs=pl.BlockSpec((_QB, H), _OUT_SPEC_F),
        scratch_shapes=list(_SCRATCH),
    )
    return pl.pallas_call(
        functools.partial(_layer_body, False),
        grid_spec=grid_spec,
        out_shape=jax.ShapeDtypeStruct((S, H), jnp.float32),
    )(idx_flat, val_flat, x, wqkv, bqkv, mask_f, *weights)


def _layer_embed(ids, tab, pos, tid, te, ge, be,
                 wqkv, bqkv, mask_f, idx_flat, val_flat, *weights):
    grid_spec = pltpu.PrefetchScalarGridSpec(
        num_scalar_prefetch=3,
        grid=(_STEPS,),
        in_specs=[
            pl.BlockSpec(memory_space=pl.ANY),
            pl.BlockSpec((_QB, H), _xmap),
            pl.BlockSpec((_QB, 1), _xmap),
            pl.BlockSpec((TV, H), lambda s, *_: (0, 0)),
            pl.BlockSpec((1, H), lambda s, *_: (0, 0)),
            pl.BlockSpec((1, H), lambda s, *_: (0, 0)),
        ] + _WSPECS,
        out_specs=pl.BlockSpec((_QB, H), _OUT_SPEC_F),
        scratch_shapes=list(_SCRATCH) + [
            pltpu.VMEM((S, H), jnp.float32),
            pltpu.SemaphoreType.DMA,
        ],
    )
    return pl.pallas_call(
        functools.partial(_layer_body, True),
        grid_spec=grid_spec,
        out_shape=jax.ShapeDtypeStruct((S, H), jnp.float32),
    )(idx_flat, val_flat, ids, tab, pos, tid, te, ge, be,
      wqkv, bqkv, mask_f, *weights)


def kernel(word_ids, mask, type_ids, word_emb, pos_emb, type_emb, ln_emb_g,
           ln_emb_b, Wq, bq, Wk, bk, Wv, bv, Wo, bo, ln1_g, ln1_b, W1, b1,
           W2, b2, ln2_g, ln2_b):
    mask_f = mask.reshape(NB, 1, BS).astype(jnp.float32)
    x = None
    for l in range(L):
        idx, valid = _LAYOUTS[l]
        idx_flat = jnp.asarray(idx.reshape(-1), jnp.int32)
        val_flat = jnp.asarray(valid.reshape(-1).astype(np.int32))
        wqkv = jnp.concatenate(
            [Wq[l], Wk[l], Wv[l]], axis=1).astype(jnp.bfloat16)
        bqkv = jnp.concatenate([bq[l], bk[l], bv[l]]).reshape(1, 3 * H)
        weights = (
            Wo[l].astype(jnp.bfloat16), bo[l].reshape(1, H),
            ln1_g[l].reshape(1, H), ln1_b[l].reshape(1, H),
            W1[l].astype(jnp.bfloat16), b1[l].reshape(1, FF),
            W2[l].astype(jnp.bfloat16), b2[l].reshape(1, H),
            ln2_g[l].reshape(1, H), ln2_b[l].reshape(1, H))
        if l == 0:
            x = _layer_embed(
                word_ids.reshape(S), word_emb, pos_emb,
                type_ids.reshape(S, 1), type_emb,
                ln_emb_g.reshape(1, H), ln_emb_b.reshape(1, H),
                wqkv, bqkv, mask_f, idx_flat, val_flat, *weights)
        else:
            x = _layer(x, wqkv, bqkv, mask_f, idx_flat, val_flat, *weights)
    return x.reshape(B, S, H)


# speedup vs baseline: 1.0184x; 1.0184x over previous
import functools

import jax
import jax.numpy as jnp
import numpy as np
from jax.experimental import pallas as pl
from jax.experimental.pallas import tpu as pltpu

B, S, H, L, NH, DH = 1, 2048, 768, 2, 12, 64
V, TV, FF, BS, NR = 30522, 2, 3072, 64, 3
NB = S // BS
NK = 5 + NR


def _block_layout(nb, num_rand, seed):
    rng = np.random.RandomState(seed)
    idx = np.zeros((nb, 5 + num_rand), np.int32)
    valid = np.ones((nb, 5 + num_rand), np.float32)
    for i in range(nb):
        fixed = [0, nb - 1, (i - 1) % nb, i, (i + 1) % nb]
        rem = sorted(set(range(nb)) - set(fixed))
        r = rng.choice(rem, num_rand, replace=False)
        row = fixed + list(r)
        seen = set()
        for j, c in enumerate(row):
            idx[i, j] = c
            if c in seen:
                valid[i, j] = 0.0
            seen.add(c)
    return idx, valid


_LAYOUTS = [_block_layout(NB, NR, i) for i in range(L)]

def _ln(x, g, b):
    m = jnp.mean(x, -1, keepdims=True)
    v = jnp.mean((x - m) * (x - m), -1, keepdims=True)
    return (x - m) / jnp.sqrt(v + 1e-12) * g + b


_QB = 512
_NQ = S // _QB
_AB = 4
_NA = NB // _AB
_STEPS = _NQ + _NA + _NQ


def _layer_body(embed, *refs):
    if embed:
        (idx_ref, val_ref, ids_ref, tab_ref, pos_ref, tid_ref, te_ref,
         ge_ref, be_ref, wqkv_ref, bqkv_ref, mask_ref, wo_ref, bo_ref,
         g1_ref, b1_ref, w1_ref, bb1_ref, w2_ref, bb2_ref, g2_ref, b2_ref,
         out_ref, q_scr, kt_scr, v_scr, kgt_scr, vg_scr,
         x_scr, sem) = refs
    else:
        (idx_ref, val_ref, x_in_ref, wqkv_ref, bqkv_ref, mask_ref,
         wo_ref, bo_ref, g1_ref, b1_ref, w1_ref, bb1_ref, w2_ref, bb2_ref,
         g2_ref, b2_ref, out_ref, q_scr, kt_scr, v_scr, kgt_scr,
         vg_scr) = refs
    step = pl.program_id(0)

    if embed:
        @pl.when(step == 0)
        def _():
            def issue(t, _):
                for u in range(8):
                    pltpu.make_async_copy(
                        tab_ref.at[pl.ds(ids_ref[t * 8 + u], 1), :],
                        x_scr.at[pl.ds(t * 8 + u, 1), :],
                        sem,
                    ).start()
                return 0

            jax.lax.fori_loop(0, S // 8, issue, 0)

    @pl.when(step < _NQ)
    def _():
        i = step
        if embed:
            def wait16(t, _):
                for _u in range(16):
                    pltpu.make_async_copy(
                        tab_ref.at[pl.ds(0, 1), :],
                        x_scr.at[pl.ds(0, 1), :], sem,
                    ).wait()
                return 0

            jax.lax.fori_loop(0, _QB // 16, wait16, 0)
            raw = x_scr[pl.ds(i * _QB, _QB), :] + pos_ref[...]
            cond = tid_ref[...] == 0
            raw = raw + jnp.where(cond, te_ref[0:1, :], te_ref[1:2, :])
            xb_f32 = _ln(raw, ge_ref[...], be_ref[...])
            x_scr[pl.ds(i * _QB, _QB), :] = xb_f32
        else:
            xb_f32 = x_in_ref[...]
        xb = xb_f32.astype(jnp.bfloat16)
        r = jax.lax.dot_general(
            xb, wqkv_ref[...], (((1,), (0,)), ((), ())),
            preferred_element_type=jnp.float32)
        r = (r + bqkv_ref[...]).astype(jnp.bfloat16)
        for h in range(NH):
            q_scr[h, pl.ds(i * _QB, _QB), :] = r[:, h * DH:(h + 1) * DH]
            v_scr[h, pl.ds(i * _QB, _QB), :] = (
                r[:, 2 * H + h * DH:2 * H + (h + 1) * DH])
            for sb in range(_QB // BS):
                kt_scr[pl.ds(i * (_QB // BS) + sb, 1), h] = jnp.transpose(
                    r[sb * BS:(sb + 1) * BS,
                      H + h * DH:H + (h + 1) * DH])[None]

    @pl.when(jnp.logical_and(step >= _NQ, step < _NQ + _NA))
    def _():
        for local in range(_AB):
            n = (step - _NQ) * _AB + local
            bias_parts = []
            for j in range(NK):
                bi = idx_ref[n * NK + j]
                kgt_scr[local, :, :, j * BS:(j + 1) * BS] = kt_scr[bi]
                vg_scr[local, :, j * BS:(j + 1) * BS, :] = (
                    v_scr[:, pl.ds(bi * BS, BS), :])
                mv = mask_ref[bi]
                vj = val_ref[n * NK + j].astype(jnp.float32)
                bias_parts.append((1.0 - mv * vj) * (-1e9))
            bias = jnp.concatenate(bias_parts, axis=-1)[None]

            qb = q_scr[:, pl.ds(n * BS, BS), :]
            s = jax.lax.dot_general(
                qb, kgt_scr[local], (((2,), (1,)), ((0,), (0,))),
                preferred_element_type=jnp.float32)
            e = jnp.exp(s * 0.125 + bias)
            denom = jnp.sum(e, -1, keepdims=True)
            o3 = jax.lax.dot_general(
                e.astype(jnp.bfloat16), vg_scr[local],
                (((2,), (1,)), ((0,), (0,))),
                preferred_element_type=jnp.float32)
            o3 = o3 * (1.0 / denom)
            for h in range(NH):
                q_scr[h, pl.ds(n * BS, BS), :] = o3[h].astype(jnp.bfloat16)

    @pl.when(step >= _NQ + _NA)
    def _():
        i = step - (_NQ + _NA)
        ob = jnp.concatenate(
            [q_scr[h, pl.ds(i * _QB, _QB), :] for h in range(NH)], axis=1)
        a = jax.lax.dot_general(
            ob, wo_ref[...], (((1,), (0,)), ((), ())),
            preferred_element_type=jnp.float32)
        if embed:
            xres = x_scr[pl.ds(i * _QB, _QB), :]
        else:
            xres = x_in_ref[...]
        a = a + bo_ref[...] + xres
        x1 = _ln(a, g1_ref[...], b1_ref[...])
        h1 = jax.lax.dot_general(
            x1.astype(jnp.bfloat16), w1_ref[...], (((1,), (0,)), ((), ())),
            preferred_element_type=jnp.float32)
        h1 = jax.nn.gelu(h1 + bb1_ref[...])
        f = jax.lax.dot_general(
            h1.astype(jnp.bfloat16), w2_ref[...], (((1,), (0,)), ((), ())),
            preferred_element_type=jnp.float32)
        f = f + bb2_ref[...] + x1
        out_ref[...] = _ln(f, g2_ref[...], b2_ref[...])


def _xmap(s, *_):
    return (jnp.where(s < _NQ, s,
                      jnp.where(s >= _NQ + _NA, s - (_NQ + _NA), 0)), 0)


_WSPECS = [
    pl.BlockSpec((H, 3 * H), lambda s, *_: (0, 0)),
    pl.BlockSpec((1, 3 * H), lambda s, *_: (0, 0)),
    pl.BlockSpec((NB, 1, BS), lambda s, *_: (0, 0, 0)),
    pl.BlockSpec((H, H), lambda s, *_: (0, 0)),
    pl.BlockSpec((1, H), lambda s, *_: (0, 0)),
    pl.BlockSpec((1, H), lambda s, *_: (0, 0)),
    pl.BlockSpec((1, H), lambda s, *_: (0, 0)),
    pl.BlockSpec((H, FF), lambda s, *_: (0, 0)),
    pl.BlockSpec((1, FF), lambda s, *_: (0, 0)),
    pl.BlockSpec((FF, H), lambda s, *_: (0, 0)),
    pl.BlockSpec((1, H), lambda s, *_: (0, 0)),
    pl.BlockSpec((1, H), lambda s, *_: (0, 0)),
    pl.BlockSpec((1, H), lambda s, *_: (0, 0)),
]

_OUT_SPEC_F = lambda s, *_: (jnp.where(s >= _NQ + _NA, s - (_NQ + _NA), 0), 0)

_SCRATCH = [
    pltpu.VMEM((NH, S, DH), jnp.bfloat16),
    pltpu.VMEM((NB, NH, DH, BS), jnp.bfloat16),
    pltpu.VMEM((NH, S, DH), jnp.bfloat16),
    pltpu.VMEM((_AB, NH, DH, NK * BS), jnp.bfloat16),
    pltpu.VMEM((_AB, NH, NK * BS, DH), jnp.bfloat16),
]


def _layer(x, wqkv, bqkv, mask_f, idx_flat, val_flat, *weights):
    grid_spec = pltpu.PrefetchScalarGridSpec(
        num_scalar_prefetch=2,
        grid=(_STEPS,),
        in_specs=[pl.BlockSpec((_QB, H), _xmap)] + _WSPECS,
        out_specs=pl.BlockSpec((_QB, H), _OUT_SPEC_F),
        scratch_shapes=list(_SCRATCH),
    )
    return pl.pallas_call(
        functools.partial(_layer_body, False),
        grid_spec=grid_spec,
        out_shape=jax.ShapeDtypeStruct((S, H), jnp.float32),
    )(idx_flat, val_flat, x, wqkv, bqkv, mask_f, *weights)


def _layer_embed(ids, tab, pos, tid, te, ge, be,
                 wqkv, bqkv, mask_f, idx_flat, val_flat, *weights):
    grid_spec = pltpu.PrefetchScalarGridSpec(
        num_scalar_prefetch=3,
        grid=(_STEPS,),
        in_specs=[
            pl.BlockSpec(memory_space=pl.ANY),
            pl.BlockSpec((_QB, H), _xmap),
            pl.BlockSpec((_QB, 1), _xmap),
            pl.BlockSpec((TV, H), lambda s, *_: (0, 0)),
            pl.BlockSpec((1, H), lambda s, *_: (0, 0)),
            pl.BlockSpec((1, H), lambda s, *_: (0, 0)),
        ] + _WSPECS,
        out_specs=pl.BlockSpec((_QB, H), _OUT_SPEC_F),
        scratch_shapes=list(_SCRATCH) + [
            pltpu.VMEM((S, H), jnp.float32),
            pltpu.SemaphoreType.DMA,
        ],
    )
    return pl.pallas_call(
        functools.partial(_layer_body, True),
        grid_spec=grid_spec,
        out_shape=jax.ShapeDtypeStruct((S, H), jnp.float32),
    )(idx_flat, val_flat, ids, tab, pos, tid, te, ge, be,
      wqkv, bqkv, mask_f, *weights)


def kernel(word_ids, mask, type_ids, word_emb, pos_emb, type_emb, ln_emb_g,
           ln_emb_b, Wq, bq, Wk, bk, Wv, bv, Wo, bo, ln1_g, ln1_b, W1, b1,
           W2, b2, ln2_g, ln2_b):
    mask_f = mask.reshape(NB, 1, BS).astype(jnp.float32)
    x = None
    for l in range(L):
        idx, valid = _LAYOUTS[l]
        idx_flat = jnp.asarray(idx.reshape(-1), jnp.int32)
        val_flat = jnp.asarray(valid.reshape(-1).astype(np.int32))
        wqkv = jnp.concatenate(
            [Wq[l], Wk[l], Wv[l]], axis=1).astype(jnp.bfloat16)
        bqkv = jnp.concatenate([bq[l], bk[l], bv[l]]).reshape(1, 3 * H)
        weights = (
            Wo[l].astype(jnp.bfloat16), bo[l].reshape(1, H),
            ln1_g[l].reshape(1, H), ln1_b[l].reshape(1, H),
            W1[l].astype(jnp.bfloat16), b1[l].reshape(1, FF),
            W2[l].astype(jnp.bfloat16), b2[l].reshape(1, H),
            ln2_g[l].reshape(1, H), ln2_b[l].reshape(1, H))
        if l == 0:
            x = _layer_embed(
                word_ids.reshape(S), word_emb, pos_emb,
                type_ids.reshape(S, 1), type_emb,
                ln_emb_g.reshape(1, H), ln_emb_b.reshape(1, H),
                wqkv, bqkv, mask_f, idx_flat, val_flat, *weights)
        else:
            x = _layer(x, wqkv, bqkv, mask_f, idx_flat, val_flat, *weights)
    return x.reshape(B, S, H)


# speedup vs baseline: 1.0287x; 1.0101x over previous
import functools

import jax
import jax.numpy as jnp
import numpy as np
from jax.experimental import pallas as pl
from jax.experimental.pallas import tpu as pltpu

B, S, H, L, NH, DH = 1, 2048, 768, 2, 12, 64
V, TV, FF, BS, NR = 30522, 2, 3072, 64, 3
NB = S // BS
NK = 5 + NR


def _block_layout(nb, num_rand, seed):
    rng = np.random.RandomState(seed)
    idx = np.zeros((nb, 5 + num_rand), np.int32)
    valid = np.ones((nb, 5 + num_rand), np.float32)
    for i in range(nb):
        fixed = [0, nb - 1, (i - 1) % nb, i, (i + 1) % nb]
        rem = sorted(set(range(nb)) - set(fixed))
        r = rng.choice(rem, num_rand, replace=False)
        row = fixed + list(r)
        seen = set()
        for j, c in enumerate(row):
            idx[i, j] = c
            if c in seen:
                valid[i, j] = 0.0
            seen.add(c)
    return idx, valid


_LAYOUTS = [_block_layout(NB, NR, i) for i in range(L)]

def _ln(x, g, b):
    m = jnp.mean(x, -1, keepdims=True)
    v = jnp.mean((x - m) * (x - m), -1, keepdims=True)
    return (x - m) * jax.lax.rsqrt(v + 1e-12) * g + b


_QB = 512
_NQ = S // _QB
_AB = 4
_NA = NB // _AB
_STEPS = _NQ + _NA + _NQ


def _layer_body(embed, *refs):
    if embed:
        (idx_ref, val_ref, ids_ref, tab_ref, pos_ref, tid_ref, te_ref,
         ge_ref, be_ref, wqkv_ref, bqkv_ref, mask_ref, wo_ref, bo_ref,
         g1_ref, b1_ref, w1_ref, bb1_ref, w2_ref, bb2_ref, g2_ref, b2_ref,
         out_ref, q_scr, kt_scr, v_scr, kgt_scr, vg_scr,
         x_scr, sem) = refs
    else:
        (idx_ref, val_ref, x_in_ref, wqkv_ref, bqkv_ref, mask_ref,
         wo_ref, bo_ref, g1_ref, b1_ref, w1_ref, bb1_ref, w2_ref, bb2_ref,
         g2_ref, b2_ref, out_ref, q_scr, kt_scr, v_scr, kgt_scr,
         vg_scr) = refs
    step = pl.program_id(0)

    if embed:
        @pl.when(step == 0)
        def _():
            def issue(t, _):
                for u in range(8):
                    pltpu.make_async_copy(
                        tab_ref.at[pl.ds(ids_ref[t * 8 + u], 1), :],
                        x_scr.at[pl.ds(t * 8 + u, 1), :],
                        sem,
                    ).start()
                return 0

            jax.lax.fori_loop(0, S // 8, issue, 0)

    @pl.when(step < _NQ)
    def _():
        i = step
        if embed:
            def wait16(t, _):
                for _u in range(16):
                    pltpu.make_async_copy(
                        tab_ref.at[pl.ds(0, 1), :],
                        x_scr.at[pl.ds(0, 1), :], sem,
                    ).wait()
                return 0

            jax.lax.fori_loop(0, _QB // 16, wait16, 0)
            raw = x_scr[pl.ds(i * _QB, _QB), :] + pos_ref[...]
            cond = tid_ref[...] == 0
            raw = raw + jnp.where(cond, te_ref[0:1, :], te_ref[1:2, :])
            xb_f32 = _ln(raw, ge_ref[...], be_ref[...])
            x_scr[pl.ds(i * _QB, _QB), :] = xb_f32
        else:
            xb_f32 = x_in_ref[...]
        xb = xb_f32.astype(jnp.bfloat16)
        r = jax.lax.dot_general(
            xb, wqkv_ref[...], (((1,), (0,)), ((), ())),
            preferred_element_type=jnp.float32)
        r = (r + bqkv_ref[...]).astype(jnp.bfloat16)
        for h in range(NH):
            q_scr[h, pl.ds(i * _QB, _QB), :] = r[:, h * DH:(h + 1) * DH]
            v_scr[h, pl.ds(i * _QB, _QB), :] = (
                r[:, 2 * H + h * DH:2 * H + (h + 1) * DH])
            for sb in range(_QB // BS):
                kt_scr[pl.ds(i * (_QB // BS) + sb, 1), h] = jnp.transpose(
                    r[sb * BS:(sb + 1) * BS,
                      H + h * DH:H + (h + 1) * DH])[None]

    @pl.when(jnp.logical_and(step >= _NQ, step < _NQ + _NA))
    def _():
        for local in range(_AB):
            n = (step - _NQ) * _AB + local
            bias_parts = []
            for j in range(NK):
                bi = idx_ref[n * NK + j]
                kgt_scr[local, :, :, j * BS:(j + 1) * BS] = kt_scr[bi]
                vg_scr[local, :, j * BS:(j + 1) * BS, :] = (
                    v_scr[:, pl.ds(bi * BS, BS), :])
                mv = mask_ref[bi]
                vj = val_ref[n * NK + j].astype(jnp.float32)
                bias_parts.append((1.0 - mv * vj) * (-1e9))
            bias = jnp.concatenate(bias_parts, axis=-1)[None]

            qb = q_scr[:, pl.ds(n * BS, BS), :]
            s = jax.lax.dot_general(
                qb, kgt_scr[local], (((2,), (1,)), ((0,), (0,))),
                preferred_element_type=jnp.float32)
            e = jnp.exp(s * 0.125 + bias)
            denom = jnp.sum(e, -1, keepdims=True)
            o3 = jax.lax.dot_general(
                e.astype(jnp.bfloat16), vg_scr[local],
                (((2,), (1,)), ((0,), (0,))),
                preferred_element_type=jnp.float32)
            o3 = o3 * (1.0 / denom)
            for h in range(NH):
                q_scr[h, pl.ds(n * BS, BS), :] = o3[h].astype(jnp.bfloat16)

    @pl.when(step >= _NQ + _NA)
    def _():
        i = step - (_NQ + _NA)
        ob = jnp.concatenate(
            [q_scr[h, pl.ds(i * _QB, _QB), :] for h in range(NH)], axis=1)
        a = jax.lax.dot_general(
            ob, wo_ref[...], (((1,), (0,)), ((), ())),
            preferred_element_type=jnp.float32)
        if embed:
            xres = x_scr[pl.ds(i * _QB, _QB), :]
        else:
            xres = x_in_ref[...]
        a = a + bo_ref[...] + xres
        x1 = _ln(a, g1_ref[...], b1_ref[...])
        h1 = jax.lax.dot_general(
            x1.astype(jnp.bfloat16), w1_ref[...], (((1,), (0,)), ((), ())),
            preferred_element_type=jnp.float32)
        h1 = jax.nn.gelu((h1 + bb1_ref[...]).astype(jnp.bfloat16))
        f = jax.lax.dot_general(
            h1, w2_ref[...], (((1,), (0,)), ((), ())),
            preferred_element_type=jnp.float32)
        f = f + bb2_ref[...] + x1
        out_ref[...] = _ln(f, g2_ref[...], b2_ref[...])


def _xmap(s, *_):
    return (jnp.where(s < _NQ, s,
                      jnp.where(s >= _NQ + _NA, s - (_NQ + _NA), 0)), 0)


_WSPECS = [
    pl.BlockSpec((H, 3 * H), lambda s, *_: (0, 0)),
    pl.BlockSpec((1, 3 * H), lambda s, *_: (0, 0)),
    pl.BlockSpec((NB, 1, BS), lambda s, *_: (0, 0, 0)),
    pl.BlockSpec((H, H), lambda s, *_: (0, 0)),
    pl.BlockSpec((1, H), lambda s, *_: (0, 0)),
    pl.BlockSpec((1, H), lambda s, *_: (0, 0)),
    pl.BlockSpec((1, H), lambda s, *_: (0, 0)),
    pl.BlockSpec((H, FF), lambda s, *_: (0, 0)),
    pl.BlockSpec((1, FF), lambda s, *_: (0, 0)),
    pl.BlockSpec((FF, H), lambda s, *_: (0, 0)),
    pl.BlockSpec((1, H), lambda s, *_: (0, 0)),
    pl.BlockSpec((1, H), lambda s, *_: (0, 0)),
    pl.BlockSpec((1, H), lambda s, *_: (0, 0)),
]

_OUT_SPEC_F = lambda s, *_: (jnp.where(s >= _NQ + _NA, s - (_NQ + _NA), 0), 0)

_SCRATCH = [
    pltpu.VMEM((NH, S, DH), jnp.bfloat16),
    pltpu.VMEM((NB, NH, DH, BS), jnp.bfloat16),
    pltpu.VMEM((NH, S, DH), jnp.bfloat16),
    pltpu.VMEM((_AB, NH, DH, NK * BS), jnp.bfloat16),
    pltpu.VMEM((_AB, NH, NK * BS, DH), jnp.bfloat16),
]


def _layer(x, wqkv, bqkv, mask_f, idx_flat, val_flat, *weights):
    grid_spec = pltpu.PrefetchScalarGridSpec(
        num_scalar_prefetch=2,
        grid=(_STEPS,),
        in_specs=[pl.BlockSpec((_QB, H), _xmap)] + _WSPECS,
        out_specs=pl.BlockSpec((_QB, H), _OUT_SPEC_F),
        scratch_shapes=list(_SCRATCH),
    )
    return pl.pallas_call(
        functools.partial(_layer_body, False),
        grid_spec=grid_spec,
        out_shape=jax.ShapeDtypeStruct((S, H), jnp.float32),
    )(idx_flat, val_flat, x, wqkv, bqkv, mask_f, *weights)


def _layer_embed(ids, tab, pos, tid, te, ge, be,
                 wqkv, bqkv, mask_f, idx_flat, val_flat, *weights):
    grid_spec = pltpu.PrefetchScalarGridSpec(
        num_scalar_prefetch=3,
        grid=(_STEPS,),
        in_specs=[
            pl.BlockSpec(memory_space=pl.ANY),
            pl.BlockSpec((_QB, H), _xmap),
            pl.BlockSpec((_QB, 1), _xmap),
            pl.BlockSpec((TV, H), lambda s, *_: (0, 0)),
            pl.BlockSpec((1, H), lambda s, *_: (0, 0)),
            pl.BlockSpec((1, H), lambda s, *_: (0, 0)),
        ] + _WSPECS,
        out_specs=pl.BlockSpec((_QB, H), _OUT_SPEC_F),
        scratch_shapes=list(_SCRATCH) + [
            pltpu.VMEM((S, H), jnp.float32),
            pltpu.SemaphoreType.DMA,
        ],
    )
    return pl.pallas_call(
        functools.partial(_layer_body, True),
        grid_spec=grid_spec,
        out_shape=jax.ShapeDtypeStruct((S, H), jnp.float32),
    )(idx_flat, val_flat, ids, tab, pos, tid, te, ge, be,
      wqkv, bqkv, mask_f, *weights)


def kernel(word_ids, mask, type_ids, word_emb, pos_emb, type_emb, ln_emb_g,
           ln_emb_b, Wq, bq, Wk, bk, Wv, bv, Wo, bo, ln1_g, ln1_b, W1, b1,
           W2, b2, ln2_g, ln2_b):
    mask_f = mask.reshape(NB, 1, BS).astype(jnp.float32)
    x = None
    for l in range(L):
        idx, valid = _LAYOUTS[l]
        idx_flat = jnp.asarray(idx.reshape(-1), jnp.int32)
        val_flat = jnp.asarray(valid.reshape(-1).astype(np.int32))
        wqkv = jnp.concatenate(
            [Wq[l], Wk[l], Wv[l]], axis=1).astype(jnp.bfloat16)
        bqkv = jnp.concatenate([bq[l], bk[l], bv[l]]).reshape(1, 3 * H)
        weights = (
            Wo[l].astype(jnp.bfloat16), bo[l].reshape(1, H),
            ln1_g[l].reshape(1, H), ln1_b[l].reshape(1, H),
            W1[l].astype(jnp.bfloat16), b1[l].reshape(1, FF),
            W2[l].astype(jnp.bfloat16), b2[l].reshape(1, H),
            ln2_g[l].reshape(1, H), ln2_b[l].reshape(1, H))
        if l == 0:
            x = _layer_embed(
                word_ids.reshape(S), word_emb, pos_emb,
                type_ids.reshape(S, 1), type_emb,
                ln_emb_g.reshape(1, H), ln_emb_b.reshape(1, H),
                wqkv, bqkv, mask_f, idx_flat, val_flat, *weights)
        else:
            x = _layer(x, wqkv, bqkv, mask_f, idx_flat, val_flat, *weights)
    return x.reshape(B, S, H)
